# Initial kernel scaffold; baseline (speedup 1.0000x reference)
import jax
import jax.numpy as jnp
from jax.experimental import pallas as pl


def kernel(node_feat, edge_attr, params, edge_index, batch):
    raise NotImplementedError("write your pallas kernel here")


# speedup vs baseline: 23.5062x; 23.5062x over previous
import functools

import jax
import jax.numpy as jnp
import numpy as np
from jax import lax
from jax.experimental import pallas as pl
from jax.experimental.pallas import tpu as pltpu
from jax.experimental.pallas import tpu_sc as plsc

_N = 10000
_E = 160000
_DN = 128
_DE = 16
_H = 128
_FD = 2
_K = 3
_PQ = 3
_NG = 64
_NCLS = 2

_NW = 32
_NPADN = 10240
_NPAD = 10016
_EPAD = 160256
_CH = _EPAD // _NW
_NV = _CH // 16
_SUBV = (160, 153)
_SUBB = 160 * 16


def _leaky(x):
  return jnp.where(x >= 0, x, 0.01 * x)


def _mlp_n_body(x_ref, w1_ref, w2_ref, b2_ref, out_ref):
  h = jnp.dot(x_ref[...], w1_ref[...], preferred_element_type=jnp.float32)
  m = jnp.sum(h, axis=0, keepdims=True) / _N
  v = jnp.sum(h * h, axis=0, keepdims=True) / _N - m * m
  h = (h - m) * lax.rsqrt(v + 1e-5)
  h = _leaky(h)
  o = lax.dot_general(w2_ref[...], h, (((0,), (1,)), ((), ())),
                      preferred_element_type=jnp.float32)
  out_ref[...] = jnp.tanh(o + b2_ref[...]) * np.pi


def _mlp_n(x, w1, w2, b2):
  return pl.pallas_call(
      _mlp_n_body,
      out_shape=jax.ShapeDtypeStruct((_FD, _N), jnp.float32),
  )(x, w1, w2, b2)


_EBLK = 16000


def _esum_body(xt_ref, w1_ref, out_ref, acc_ref):
  i = pl.program_id(0)

  @pl.when(i == 0)
  def _():
    acc_ref[...] = jnp.zeros_like(acc_ref)

  h = lax.dot_general(w1_ref[...], xt_ref[...], (((0,), (0,)), ((), ())),
                      preferred_element_type=jnp.float32)
  acc_ref[:, 0:1] += jnp.sum(h, axis=1, keepdims=True)
  acc_ref[:, 1:2] += jnp.sum(h * h, axis=1, keepdims=True)

  @pl.when(i == _E // _EBLK - 1)
  def _():
    m = acc_ref[:, 0:1] / _E
    v = acc_ref[:, 1:2] / _E - m * m
    out_ref[:, 0:1] = m
    out_ref[:, 1:2] = lax.rsqrt(v + 1e-5)


def _esum(xt, w1):
  return pl.pallas_call(
      _esum_body,
      grid=(_E // _EBLK,),
      in_specs=[
          pl.BlockSpec((_DE, _EBLK), lambda i: (0, i)),
          pl.BlockSpec((_DE, _H), lambda i: (0, 0)),
      ],
      out_specs=pl.BlockSpec((_H, 2), lambda i: (0, 0)),
      out_shape=jax.ShapeDtypeStruct((_H, 2), jnp.float32),
      scratch_shapes=[pltpu.VMEM((_H, 2), jnp.float32)],
  )(xt, w1)


def _mlp_e_body(xt_ref, w1_ref, st_ref, w2_ref, b2_ref, out_ref):
  h = lax.dot_general(w1_ref[...], xt_ref[...], (((0,), (0,)), ((), ())),
                      preferred_element_type=jnp.float32)
  hn = _leaky((h - st_ref[:, 0:1]) * st_ref[:, 1:2])
  o = lax.dot_general(w2_ref[...], hn, (((0,), (0,)), ((), ())),
                      preferred_element_type=jnp.float32)
  out_ref[...] = jnp.tanh(o + b2_ref[...]) * np.pi


def _mlp_e(xt, w1, st, w2, b2):
  return pl.pallas_call(
      _mlp_e_body,
      grid=(_E // _EBLK,),
      in_specs=[
          pl.BlockSpec((_DE, _EBLK), lambda i: (0, i)),
          pl.BlockSpec((_DE, _H), lambda i: (0, 0)),
          pl.BlockSpec((_H, 2), lambda i: (0, 0)),
          pl.BlockSpec((_H, _FD), lambda i: (0, 0)),
          pl.BlockSpec((_FD, 1), lambda i: (0, 0)),
      ],
      out_specs=pl.BlockSpec((_FD, _EBLK), lambda i: (0, i)),
      out_shape=jax.ShapeDtypeStruct((_FD, _E), jnp.float32),
  )(xt, w1, st, w2, b2)


def _sc1_body(dst_h, zi_h, out_h, dst_v, cnt_v):
  c = lax.axis_index("c")
  s = lax.axis_index("s")
  wid = c * 16 + s
  pltpu.sync_copy(zi_h, cnt_v)
  pltpu.sync_copy(dst_h.at[pl.ds(wid * _CH, _CH)], dst_v)
  qbase = jnp.min(plsc.scan_count(jnp.zeros((16,), jnp.int32))[0])

  @pl.loop(0, _NV)
  def _(i):
    d = dst_v[pl.ds(i * 16, 16)]
    q, lastm = plsc.scan_count(d)
    cc = plsc.load_gather(cnt_v, [d])
    plsc.store_scatter(cnt_v, [d], cc + (q - qbase) + 1, mask=lastm)

  pltpu.sync_copy(cnt_v, out_h.at[wid])


def _sc_counts(dstp, zi):
  mesh = plsc.VectorSubcoreMesh(core_axis_name="c", subcore_axis_name="s")
  f = pl.kernel(
      _sc1_body,
      out_type=jax.ShapeDtypeStruct((_NW, _NPAD), jnp.int32),
      mesh=mesh,
      compiler_params=pltpu.CompilerParams(needs_layout_passes=False,
                                           use_tc_tiling_on_sc=False),
      scratch_types=[
          pltpu.VMEM((_CH,), jnp.int32),
          pltpu.VMEM((_NPAD,), jnp.int32),
      ],
  )
  return f(dstp, zi)


def _prefix_body(cnt_ref, base_ref, tot_ref):
  x = cnt_ref[...]
  run = jnp.zeros((1, _NPAD), jnp.int32)
  rows = []
  for i in range(_NW):
    rows.append(run)
    run = run + x[i:i + 1]
  base_ref[...] = jnp.concatenate(rows, axis=0)
  tot_ref[...] = run


def _prefix(cnt_all):
  return pl.pallas_call(
      _prefix_body,
      out_shape=(jax.ShapeDtypeStruct((_NW, _NPAD), jnp.int32),
                 jax.ShapeDtypeStruct((1, _NPAD), jnp.int32)),
  )(cnt_all)


def _sc2_body(dst_h, src_h, ef0_h, ef1_h, nf0_h, nf1_h, base_h, wtab_h, zf_h,
              out_h, cnt_v, nf0_v, nf1_v, dst_v, src_v, e0_v, e1_v,
              we0_v, we1_v, wn0_v, wn1_v, acc0_v, acc1_v, acc2_v):
  c = lax.axis_index("c")
  s = lax.axis_index("s")
  wid = c * 16 + s
  pltpu.sync_copy(base_h.at[wid], cnt_v)
  pltpu.sync_copy(nf0_h, nf0_v)
  pltpu.sync_copy(nf1_h, nf1_v)
  pltpu.sync_copy(wtab_h.at[0], we0_v)
  pltpu.sync_copy(wtab_h.at[1], we1_v)
  pltpu.sync_copy(wtab_h.at[2], wn0_v)
  pltpu.sync_copy(wtab_h.at[3], wn1_v)
  pltpu.sync_copy(zf_h, acc0_v)
  pltpu.sync_copy(zf_h, acc1_v)
  pltpu.sync_copy(zf_h, acc2_v)
  qbase = jnp.min(plsc.scan_count(jnp.zeros((16,), jnp.int32))[0])
  ebase = wid * _CH

  accs = (acc0_v, acc1_v, acc2_v)
  wes = ((we0_v, we1_v), (wn0_v, wn1_v))
  voff = 0
  for nv in _SUBV:
    nb = nv * 16
    wb = ebase + voff * 16
    pltpu.sync_copy(dst_h.at[pl.ds(wb, nb)], dst_v.at[pl.ds(0, nb)])
    pltpu.sync_copy(src_h.at[pl.ds(wb, nb)], src_v.at[pl.ds(0, nb)])
    pltpu.sync_copy(ef0_h.at[pl.ds(wb, nb)], e0_v.at[pl.ds(0, nb)])
    pltpu.sync_copy(ef1_h.at[pl.ds(wb, nb)], e1_v.at[pl.ds(0, nb)])

    @pl.loop(0, nv)
    def _(i):
      d = dst_v[pl.ds(i * 16, 16)]
      sv = src_v[pl.ds(i * 16, 16)]
      q, lastm = plsc.scan_count(d)
      r = q - qbase
      cc = plsc.load_gather(cnt_v, [d])
      slot = cc + r
      plsc.store_scatter(cnt_v, [d], slot + 1, mask=lastm)
      keep = (slot < _K) & (d < _N)
      slotc = jnp.minimum(slot, _K - 1)
      kidx = slotc * _PQ
      e0 = e0_v[pl.ds(i * 16, 16)]
      e1 = e1_v[pl.ds(i * 16, 16)]
      n0 = plsc.load_gather(nf0_v, [sv])
      n1 = plsc.load_gather(nf1_v, [sv])
      for j in range(_PQ):
        w0 = plsc.load_gather(wes[0][0], [kidx + j])
        w1 = plsc.load_gather(wes[0][1], [kidx + j])
        w2 = plsc.load_gather(wes[1][0], [kidx + j])
        w3 = plsc.load_gather(wes[1][1], [kidx + j])
        contrib = e0 * w0 + e1 * w1 + n0 * w2 + n1 * w3
        plsc.store_scatter(accs[j], [slotc, d], contrib, mask=keep)

    voff += nv

  for j in range(_PQ):
    pltpu.sync_copy(accs[j], out_h.at[wid, j])


def _sc_contribs(dstp, srcp, ef0p, ef1p, nf0, nf1, base_all, wtab, zf):
  mesh = plsc.VectorSubcoreMesh(core_axis_name="c", subcore_axis_name="s")
  f = pl.kernel(
      _sc2_body,
      out_type=jax.ShapeDtypeStruct((_NW, _PQ, _K, _N), jnp.float32),
      mesh=mesh,
      compiler_params=pltpu.CompilerParams(needs_layout_passes=False,
                                           use_tc_tiling_on_sc=False),
      scratch_types=[
          pltpu.VMEM((_NPAD,), jnp.int32),
          pltpu.VMEM((_N,), jnp.float32),
          pltpu.VMEM((_N,), jnp.float32),
          pltpu.VMEM((_SUBB,), jnp.int32),
          pltpu.VMEM((_SUBB,), jnp.int32),
          pltpu.VMEM((_SUBB,), jnp.float32),
          pltpu.VMEM((_SUBB,), jnp.float32),
          pltpu.VMEM((16,), jnp.float32),
          pltpu.VMEM((16,), jnp.float32),
          pltpu.VMEM((16,), jnp.float32),
          pltpu.VMEM((16,), jnp.float32),
          pltpu.VMEM((_K, _N), jnp.float32),
          pltpu.VMEM((_K, _N), jnp.float32),
          pltpu.VMEM((_K, _N), jnp.float32),
      ],
  )
  return f(dstp, srcp, ef0p, ef1p, nf0, nf1, base_all, wtab, zf)


def _final_body(acc_ref, nf_ref, tot_ref, batch_ref, wq_ref,
                wu1_ref, bu1_ref, wu2_ref, bu2_ref, lng_ref, lnb_ref,
                wh1_ref, bh1_ref, wh2_ref, bh2_ref, wh3_ref, bh3_ref,
                out_ref):
  res = jnp.sum(acc_ref[...], axis=0)
  nf_t = nf_ref[...]
  wq = wq_ref[...]
  center = lax.dot_general(wq[2 * _K:2 * _K + _FD], nf_t,
                           (((0,), (0,)), ((), ())),
                           preferred_element_type=jnp.float32)
  acc_t = jnp.sum(res, axis=1)
  msg_t = jnp.cos(acc_t + center)
  x5 = jnp.concatenate([nf_t, msg_t], axis=0)
  h = lax.dot_general(wu1_ref[...], x5, (((0,), (0,)), ((), ())),
                      preferred_element_type=jnp.float32) + bu1_ref[...]
  h = _leaky(h)
  upd = lax.dot_general(wu2_ref[...], h, (((0,), (0,)), ((), ())),
                        preferred_element_type=jnp.float32) + bu2_ref[...]
  has_in = (tot_ref[...] > 0).astype(jnp.float32)
  un = upd * has_in
  m = (un[0:1] + un[1:2]) * 0.5
  v = ((un[0:1] - m) ** 2 + (un[1:2] - m) ** 2) * 0.5
  nf2 = (un - m) * lax.rsqrt(v + 1e-5) * lng_ref[...] + lnb_ref[...] + nf_t

  rows = lax.broadcasted_iota(jnp.int32, (_NG, _N), 0)
  oh = (rows == batch_ref[...]).astype(jnp.float32)
  g_t = lax.dot_general(nf2, oh, (((1,), (1,)), ((), ())),
                        preferred_element_type=jnp.float32,
                        precision=lax.Precision.HIGHEST)
  h1 = lax.dot_general(g_t, wh1_ref[...], (((0,), (0,)), ((), ())),
                       preferred_element_type=jnp.float32) + bh1_ref[...]
  m1 = jnp.mean(h1, axis=0, keepdims=True)
  v1 = jnp.mean(h1 * h1, axis=0, keepdims=True) - m1 * m1
  h1 = _leaky((h1 - m1) * lax.rsqrt(v1 + 1e-5))
  h2 = jnp.dot(h1, wh2_ref[...],
               preferred_element_type=jnp.float32) + bh2_ref[...]
  m2 = jnp.mean(h2, axis=0, keepdims=True)
  v2 = jnp.mean(h2 * h2, axis=0, keepdims=True) - m2 * m2
  h2 = _leaky((h2 - m2) * lax.rsqrt(v2 + 1e-5))
  out_ref[...] = jnp.dot(h2, wh3_ref[...],
                         preferred_element_type=jnp.float32) + bh3_ref[...]


def _final(acc_all, nf_t, tot, batch2d, wq, wu1, bu1, wu2, bu2, lng, lnb,
           wh1, bh1, wh2, bh2, wh3, bh3):
  return pl.pallas_call(
      _final_body,
      out_shape=jax.ShapeDtypeStruct((_NG, _NCLS), jnp.float32),
  )(acc_all, nf_t, tot, batch2d, wq, wu1, bu1, wu2, bu2, lng, lnb,
    wh1, bh1, wh2, bh2, wh3, bh3)


def kernel(node_feat, edge_attr, params, edge_index, batch):
  p = params
  f32 = jnp.float32

  eat = edge_attr.T
  bn2 = p['bn2'].reshape(_FD, 1).astype(f32)
  be2 = p['be2'].reshape(_FD, 1).astype(f32)
  wn1 = p['Wn1'].astype(f32)
  wn2 = p['Wn2'].astype(f32)
  we1 = p['We1'].astype(f32)
  we2 = p['We2'].astype(f32)
  wq = p['Wq'].astype(f32)

  nf_tp = _mlp_n(node_feat.astype(f32), wn1, wn2, bn2)
  est = _esum(eat, we1)
  ef_t = _mlp_e(eat, we1, est, we2, be2)

  dst = edge_index[1]
  src = edge_index[0]
  def _rbf(x):
    return x.astype(jnp.bfloat16).astype(f32)
  dstp = jnp.concatenate([dst, jnp.full((_EPAD - _E,), _N, jnp.int32)])
  srcp = jnp.concatenate([src, jnp.zeros((_EPAD - _E,), jnp.int32)])
  ef0p = _rbf(jnp.concatenate([ef_t[0], jnp.zeros((_EPAD - _E,), f32)]))
  ef1p = _rbf(jnp.concatenate([ef_t[1], jnp.zeros((_EPAD - _E,), f32)]))
  nf0 = _rbf(nf_tp[0, :_N])
  nf1 = _rbf(nf_tp[1, :_N])
  nf_t = nf_tp[:, :_N]

  zi = jnp.zeros((_NPAD,), jnp.int32)
  zf = jnp.zeros((_K, _N), f32)
  wtab = jnp.zeros((4, 16), f32)
  wtab = wtab.at[0, :9].set(wq[0:2 * _K:2].reshape(-1))
  wtab = wtab.at[1, :9].set(wq[1:2 * _K:2].reshape(-1))
  wtab = wtab.at[2, :9].set(wq[2 * _K + _FD::2].reshape(-1))
  wtab = wtab.at[3, :9].set(wq[2 * _K + _FD + 1::2].reshape(-1))
  wtab = _rbf(wtab)

  cnt_all = _sc_counts(dstp, zi)
  base_all, tot = _prefix(cnt_all)
  acc_all = _sc_contribs(dstp, srcp, ef0p, ef1p, nf0, nf1, base_all, wtab, zf)

  out = _final(
      acc_all, nf_t, tot[:, :_N], batch.reshape(1, _N),
      wq,
      p['Wu1'].astype(f32), p['bu1'].reshape(_H, 1).astype(f32),
      p['Wu2'].astype(f32), p['bu2'].reshape(_FD, 1).astype(f32),
      p['ln_g'].reshape(_FD, 1).astype(f32),
      p['ln_b'].reshape(_FD, 1).astype(f32),
      p['Wh1'].astype(f32), p['bh1'].reshape(1, _H).astype(f32),
      p['Wh2'].astype(f32), p['bh2'].reshape(1, _H).astype(f32),
      p['Wh3'].astype(f32), p['bh3'].reshape(1, _NCLS).astype(f32),
  )
  return out
